# baseline (speedup 1.0000x reference)
import functools

import jax
import jax.numpy as jnp
from jax import lax
from jax.experimental import pallas as pl
from jax.experimental.pallas import tpu as pltpu
from jax.experimental.pallas import tpu_sc as plsc

_f32 = jnp.float32


def _make_spmm(nf, n, e, chunk=6400):
    ncores, nsub, lanes = 2, 16, 16
    fpt = nf // nsub
    eshard = e // ncores
    nch = eshard // chunk
    groups = chunk // lanes
    assert eshard % chunk == 0 and chunk % 128 == 0 and n % lanes == 0

    mesh = plsc.VectorSubcoreMesh(core_axis_name="c", subcore_axis_name="s")

    @functools.partial(
        pl.kernel,
        out_type=jax.ShapeDtypeStruct((ncores, nf, n), _f32),
        mesh=mesh,
        compiler_params=pltpu.CompilerParams(needs_layout_passes=False),
        scratch_types=[
            pltpu.VMEM((fpt, n), _f32),
            pltpu.VMEM((fpt, n), _f32),
            pltpu.VMEM((2, 2 * chunk), jnp.int32),
            pltpu.VMEM((2 * chunk,), _f32),
            pltpu.SemaphoreType.DMA,
            pltpu.SemaphoreType.DMA,
        ],
    )
    def spmm(ht_hbm, ei_hbm, ew_hbm, out_hbm, src_v, acc_v, rc_b, w_b,
             sem0, sem1):
        c = lax.axis_index("c")
        s = lax.axis_index("s")
        f0 = s * fpt
        ebase = c * eshard

        pltpu.sync_copy(ht_hbm.at[pl.ds(f0, fpt)], src_v)

        zeros16 = jnp.zeros((lanes,), _f32)

        @pl.loop(0, n // lanes, unroll=8)
        def _zero(i):
            for r in range(fpt):
                acc_v[r, pl.ds(i * lanes, lanes)] = zeros16

        sems = (sem0, sem1)
        descs = {}

        def issue(ci, b):
            base = ebase + ci * chunk
            dst = pl.ds(b * chunk, chunk)
            descs[ci] = (
                pltpu.async_copy(ei_hbm.at[:, pl.ds(base, chunk)],
                                 rc_b.at[:, dst], sems[b]),
                pltpu.async_copy(ew_hbm.at[pl.ds(base, chunk)],
                                 w_b.at[dst], sems[b]),
            )

        issue(0, 0)
        if nch > 1:
            issue(1, 1)

        ridx = [jnp.full((lanes,), r, jnp.int32) for r in range(fpt)]

        for ci in range(nch):
            b = ci % 2
            for d in descs.pop(ci):
                d.wait()

            @plsc.parallel_loop(0, groups, unroll=16)
            def _edges(k, b=b):
                sl = pl.ds(b * chunk + k * lanes, lanes)
                rv = rc_b[0, sl]
                cv = rc_b[1, sl]
                wv = w_b[sl]
                for r in range(fpt):
                    g = plsc.load_gather(src_v, [ridx[r], cv])
                    plsc.addupdate_scatter(acc_v, [ridx[r], rv], wv * g)

            if ci + 2 < nch:
                issue(ci + 2, b)

        pltpu.sync_copy(acc_v, out_hbm.at[c, pl.ds(f0, fpt)])

    return spmm


def _dense1(x, w1, b1r):
    n, d = x.shape
    h1 = w1.shape[1]

    def body(x_ref, w_ref, b_ref, out_ref):
        acc = lax.dot_general(w_ref[...], x_ref[...],
                              (((0,), (1,)), ((), ())),
                              preferred_element_type=_f32)
        out_ref[...] = acc + b_ref[...].reshape(h1, 1)

    return pl.pallas_call(
        body,
        out_shape=jax.ShapeDtypeStruct((h1, n), _f32),
    )(x, w1, b1r)


def _dense2(p, wmu, bmur, wls, blsr):
    _, h1, n = p.shape
    h2 = wmu.shape[1]

    def body(p_ref, wmu_ref, bmu_ref, wls_ref, bls_ref, out_ref):
        h = jnp.maximum(p_ref[0] + p_ref[1], 0.0)
        top = lax.dot_general(wmu_ref[...], h, (((0,), (0,)), ((), ())),
                              preferred_element_type=_f32)
        bot = lax.dot_general(wls_ref[...], h, (((0,), (0,)), ((), ())),
                              preferred_element_type=_f32)
        top = top + bmu_ref[...].reshape(h2, 1)
        bot = bot + bls_ref[...].reshape(h2, 1)
        out_ref[...] = jnp.concatenate([top, bot], axis=0)

    return pl.pallas_call(
        body,
        out_shape=jax.ShapeDtypeStruct((2 * h2, n), _f32),
    )(p, wmu, bmur, wls, blsr)


def _final(p, noise):
    _, nf, n = p.shape
    h2 = nf // 2

    def body(p_ref, noise_ref, z_ref, mu_ref, ls_ref):
        st = p_ref[0] + p_ref[1]
        eye = jax.lax.broadcasted_iota(jnp.int32, (nf, h2), 0)
        col = jax.lax.broadcasted_iota(jnp.int32, (nf, h2), 1)
        sel_mu = (eye == col).astype(_f32)
        sel_ls = (eye == col + h2).astype(_f32)
        mu = lax.dot_general(st, sel_mu, (((0,), (0,)), ((), ())),
                             preferred_element_type=_f32)
        ls = lax.dot_general(st, sel_ls, (((0,), (0,)), ((), ())),
                             preferred_element_type=_f32)
        mu_ref[...] = mu
        ls_ref[...] = ls
        z_ref[...] = mu + noise_ref[...] * jnp.exp(ls)

    out_sds = jax.ShapeDtypeStruct((n, h2), _f32)
    return pl.pallas_call(
        body,
        out_shape=[out_sds, out_sds, out_sds],
    )(p, noise)


def kernel(x, edge_index, edge_weight, noise, W1, b1, Wmu, bmu, Wls, bls):
    n, d = x.shape
    h1 = W1.shape[1]
    h2 = Wmu.shape[1]
    e = edge_weight.shape[0]

    spmm = _make_spmm(2 * h2, n, e)

    h0t = _dense1(x, W1, b1.reshape(1, h1))
    p1 = spmm(h0t, edge_index, edge_weight)
    pret = _dense2(p1, Wmu, bmu.reshape(1, h2), Wls, bls.reshape(1, h2))
    p2 = spmm(pret, edge_index, edge_weight)
    z, mu, logstd = _final(p2, noise)
    return (z, mu, logstd)


# speedup vs baseline: 1.0906x; 1.0906x over previous
import functools

import jax
import jax.numpy as jnp
from jax import lax
from jax.experimental import pallas as pl
from jax.experimental.pallas import tpu as pltpu
from jax.experimental.pallas import tpu_sc as plsc

_f32 = jnp.float32


def _make_spmm(nf, n, e, chunk=6400):
    ncores, nsub, lanes = 2, 16, 16
    fpt = nf // nsub
    eshard = e // ncores
    nch = eshard // chunk
    groups = chunk // lanes
    assert eshard % chunk == 0 and chunk % 128 == 0 and n % lanes == 0

    mesh = plsc.VectorSubcoreMesh(core_axis_name="c", subcore_axis_name="s")

    @functools.partial(
        pl.kernel,
        out_type=jax.ShapeDtypeStruct((ncores, nf, n), _f32),
        mesh=mesh,
        compiler_params=pltpu.CompilerParams(needs_layout_passes=False),
        scratch_types=[
            pltpu.VMEM((fpt, n), _f32),
            pltpu.VMEM((fpt, n), _f32),
            pltpu.VMEM((2, 2 * chunk), jnp.int32),
            pltpu.VMEM((2 * chunk,), _f32),
            pltpu.SemaphoreType.DMA,
            pltpu.SemaphoreType.DMA,
        ],
    )
    def spmm(ht_hbm, ei_hbm, ew_hbm, out_hbm, src_v, acc_v, rc_b, w_b,
             sem0, sem1):
        c = lax.axis_index("c")
        s = lax.axis_index("s")
        f0 = s * fpt
        ebase = c * eshard

        pltpu.sync_copy(ht_hbm.at[pl.ds(f0, fpt)], src_v)

        zeros16 = jnp.zeros((lanes,), _f32)

        @pl.loop(0, n // lanes, unroll=8)
        def _zero(i):
            for r in range(fpt):
                acc_v[r, pl.ds(i * lanes, lanes)] = zeros16

        sems = (sem0, sem1)
        descs = {}

        def issue(ci, b):
            base = ebase + ci * chunk
            dst = pl.ds(b * chunk, chunk)
            descs[ci] = (
                pltpu.async_copy(ei_hbm.at[:, pl.ds(base, chunk)],
                                 rc_b.at[:, dst], sems[b]),
                pltpu.async_copy(ew_hbm.at[pl.ds(base, chunk)],
                                 w_b.at[dst], sems[b]),
            )

        issue(0, 0)
        if nch > 1:
            issue(1, 1)

        ridx = [jnp.full((lanes,), r, jnp.int32) for r in range(fpt)]

        for ci in range(nch):
            b = ci % 2
            for d in descs.pop(ci):
                d.wait()

            @plsc.parallel_loop(0, groups, unroll=8)
            def _edges(k, b=b):
                sl = pl.ds(b * chunk + k * lanes, lanes)
                rv = rc_b[0, sl]
                cv = rc_b[1, sl]
                wv = w_b[sl]
                for r in range(fpt):
                    g = plsc.load_gather(src_v, [ridx[r], cv])
                    plsc.addupdate_scatter(acc_v, [ridx[r], rv], wv * g)

            if ci + 2 < nch:
                issue(ci + 2, b)

        pltpu.sync_copy(acc_v, out_hbm.at[c, pl.ds(f0, fpt)])

    return spmm


def _dense1(x, w1, b1r):
    n, d = x.shape
    h1 = w1.shape[1]

    def body(x_ref, w_ref, b_ref, out_ref):
        acc = lax.dot_general(w_ref[...], x_ref[...],
                              (((0,), (1,)), ((), ())),
                              preferred_element_type=_f32)
        out_ref[...] = acc + b_ref[...].reshape(h1, 1)

    return pl.pallas_call(
        body,
        out_shape=jax.ShapeDtypeStruct((h1, n), _f32),
    )(x, w1, b1r)


def _dense2(p, wmu, bmur, wls, blsr):
    _, h1, n = p.shape
    h2 = wmu.shape[1]

    def body(p_ref, wmu_ref, bmu_ref, wls_ref, bls_ref, out_ref):
        h = jnp.maximum(p_ref[0] + p_ref[1], 0.0)
        top = lax.dot_general(wmu_ref[...], h, (((0,), (0,)), ((), ())),
                              preferred_element_type=_f32)
        bot = lax.dot_general(wls_ref[...], h, (((0,), (0,)), ((), ())),
                              preferred_element_type=_f32)
        top = top + bmu_ref[...].reshape(h2, 1)
        bot = bot + bls_ref[...].reshape(h2, 1)
        out_ref[...] = jnp.concatenate([top, bot], axis=0)

    return pl.pallas_call(
        body,
        out_shape=jax.ShapeDtypeStruct((2 * h2, n), _f32),
    )(p, wmu, bmur, wls, blsr)


def _final(p, noise):
    _, nf, n = p.shape
    h2 = nf // 2

    def body(p_ref, noise_ref, z_ref, mu_ref, ls_ref):
        st = p_ref[0] + p_ref[1]
        eye = jax.lax.broadcasted_iota(jnp.int32, (nf, h2), 0)
        col = jax.lax.broadcasted_iota(jnp.int32, (nf, h2), 1)
        sel_mu = (eye == col).astype(_f32)
        sel_ls = (eye == col + h2).astype(_f32)
        mu = lax.dot_general(st, sel_mu, (((0,), (0,)), ((), ())),
                             preferred_element_type=_f32)
        ls = lax.dot_general(st, sel_ls, (((0,), (0,)), ((), ())),
                             preferred_element_type=_f32)
        mu_ref[...] = mu
        ls_ref[...] = ls
        z_ref[...] = mu + noise_ref[...] * jnp.exp(ls)

    out_sds = jax.ShapeDtypeStruct((n, h2), _f32)
    return pl.pallas_call(
        body,
        out_shape=[out_sds, out_sds, out_sds],
    )(p, noise)


def kernel(x, edge_index, edge_weight, noise, W1, b1, Wmu, bmu, Wls, bls):
    n, d = x.shape
    h1 = W1.shape[1]
    h2 = Wmu.shape[1]
    e = edge_weight.shape[0]

    spmm = _make_spmm(2 * h2, n, e)

    h0t = _dense1(x, W1, b1.reshape(1, h1))
    p1 = spmm(h0t, edge_index, edge_weight)
    pret = _dense2(p1, Wmu, bmu.reshape(1, h2), Wls, bls.reshape(1, h2))
    p2 = spmm(pret, edge_index, edge_weight)
    z, mu, logstd = _final(p2, noise)
    return (z, mu, logstd)


# speedup vs baseline: 1.0963x; 1.0052x over previous
import functools

import jax
import jax.numpy as jnp
from jax import lax
from jax.experimental import pallas as pl
from jax.experimental.pallas import tpu as pltpu
from jax.experimental.pallas import tpu_sc as plsc

_f32 = jnp.float32


def _make_spmm(nf, n, e, chunk=6400):
    ncores, nsub, lanes = 2, 16, 16
    fpt = nf // nsub
    eshard = e // ncores
    nch = eshard // chunk
    groups = chunk // lanes
    assert eshard % chunk == 0 and chunk % 128 == 0 and n % lanes == 0

    mesh = plsc.VectorSubcoreMesh(core_axis_name="c", subcore_axis_name="s")

    @functools.partial(
        pl.kernel,
        out_type=jax.ShapeDtypeStruct((ncores, nf, n), _f32),
        mesh=mesh,
        compiler_params=pltpu.CompilerParams(needs_layout_passes=False),
        scratch_types=[
            pltpu.VMEM((fpt, n), _f32),
            pltpu.VMEM((fpt, n), _f32),
            pltpu.VMEM((2, 2 * chunk), jnp.int32),
            pltpu.VMEM((2 * chunk,), _f32),
            pltpu.SemaphoreType.DMA,
            pltpu.SemaphoreType.DMA,
            pltpu.SemaphoreType.DMA,
        ],
    )
    def spmm(ht_hbm, ei_hbm, ew_hbm, out_hbm, src_v, acc_v, rc_b, w_b,
             sem0, sem1, sem_src):
        c = lax.axis_index("c")
        s = lax.axis_index("s")
        f0 = s * fpt
        ebase = c * eshard

        src_cp = pltpu.async_copy(ht_hbm.at[pl.ds(f0, fpt)], src_v, sem_src)

        sems = (sem0, sem1)
        descs = {}

        def issue(ci, b):
            base = ebase + ci * chunk
            dst = pl.ds(b * chunk, chunk)
            descs[ci] = (
                pltpu.async_copy(ei_hbm.at[:, pl.ds(base, chunk)],
                                 rc_b.at[:, dst], sems[b]),
                pltpu.async_copy(ew_hbm.at[pl.ds(base, chunk)],
                                 w_b.at[dst], sems[b]),
            )

        issue(0, 0)
        if nch > 1:
            issue(1, 1)

        zeros16 = jnp.zeros((lanes,), _f32)

        @pl.loop(0, n // lanes, unroll=8)
        def _zero(i):
            for r in range(fpt):
                acc_v[r, pl.ds(i * lanes, lanes)] = zeros16

        src_cp.wait()

        ridx = [jnp.full((lanes,), r, jnp.int32) for r in range(fpt)]

        for ci in range(nch):
            b = ci % 2
            for d in descs.pop(ci):
                d.wait()

            @plsc.parallel_loop(0, groups, unroll=8)
            def _edges(k, b=b):
                sl = pl.ds(b * chunk + k * lanes, lanes)
                rv = rc_b[0, sl]
                cv = rc_b[1, sl]
                wv = w_b[sl]
                for r in range(fpt):
                    g = plsc.load_gather(src_v, [ridx[r], cv])
                    plsc.addupdate_scatter(acc_v, [ridx[r], rv], wv * g)

            if ci + 2 < nch:
                issue(ci + 2, b)

        pltpu.sync_copy(acc_v, out_hbm.at[c, pl.ds(f0, fpt)])

    return spmm


def _dense1(x, w1, b1r):
    n, d = x.shape
    h1 = w1.shape[1]

    def body(x_ref, w_ref, b_ref, out_ref):
        acc = lax.dot_general(w_ref[...], x_ref[...],
                              (((0,), (1,)), ((), ())),
                              preferred_element_type=_f32)
        out_ref[...] = acc + b_ref[...].reshape(h1, 1)

    return pl.pallas_call(
        body,
        out_shape=jax.ShapeDtypeStruct((h1, n), _f32),
    )(x, w1, b1r)


def _dense2(p, wmu, bmur, wls, blsr):
    _, h1, n = p.shape
    h2 = wmu.shape[1]

    def body(p_ref, wmu_ref, bmu_ref, wls_ref, bls_ref, out_ref):
        h = jnp.maximum(p_ref[0] + p_ref[1], 0.0)
        top = lax.dot_general(wmu_ref[...], h, (((0,), (0,)), ((), ())),
                              preferred_element_type=_f32)
        bot = lax.dot_general(wls_ref[...], h, (((0,), (0,)), ((), ())),
                              preferred_element_type=_f32)
        top = top + bmu_ref[...].reshape(h2, 1)
        bot = bot + bls_ref[...].reshape(h2, 1)
        out_ref[...] = jnp.concatenate([top, bot], axis=0)

    return pl.pallas_call(
        body,
        out_shape=jax.ShapeDtypeStruct((2 * h2, n), _f32),
    )(p, wmu, bmur, wls, blsr)


def _final(p, noise):
    _, nf, n = p.shape
    h2 = nf // 2

    def body(p_ref, noise_ref, z_ref, mu_ref, ls_ref):
        st = p_ref[0] + p_ref[1]
        eye = jax.lax.broadcasted_iota(jnp.int32, (nf, h2), 0)
        col = jax.lax.broadcasted_iota(jnp.int32, (nf, h2), 1)
        sel_mu = (eye == col).astype(_f32)
        sel_ls = (eye == col + h2).astype(_f32)
        mu = lax.dot_general(st, sel_mu, (((0,), (0,)), ((), ())),
                             preferred_element_type=_f32)
        ls = lax.dot_general(st, sel_ls, (((0,), (0,)), ((), ())),
                             preferred_element_type=_f32)
        mu_ref[...] = mu
        ls_ref[...] = ls
        z_ref[...] = mu + noise_ref[...] * jnp.exp(ls)

    out_sds = jax.ShapeDtypeStruct((n, h2), _f32)
    return pl.pallas_call(
        body,
        out_shape=[out_sds, out_sds, out_sds],
    )(p, noise)


def kernel(x, edge_index, edge_weight, noise, W1, b1, Wmu, bmu, Wls, bls):
    n, d = x.shape
    h1 = W1.shape[1]
    h2 = Wmu.shape[1]
    e = edge_weight.shape[0]

    spmm = _make_spmm(2 * h2, n, e)

    h0t = _dense1(x, W1, b1.reshape(1, h1))
    p1 = spmm(h0t, edge_index, edge_weight)
    pret = _dense2(p1, Wmu, bmu.reshape(1, h2), Wls, bls.reshape(1, h2))
    p2 = spmm(pret, edge_index, edge_weight)
    z, mu, logstd = _final(p2, noise)
    return (z, mu, logstd)


# speedup vs baseline: 1.1821x; 1.0782x over previous
import functools

import jax
import jax.numpy as jnp
from jax import lax
from jax.experimental import pallas as pl
from jax.experimental.pallas import tpu as pltpu
from jax.experimental.pallas import tpu_sc as plsc

_f32 = jnp.float32


def _make_spmm(nf, n, e, chunk=3200, fpt=4):
    ncores, nsub, lanes = 2, 16, 16
    fgroups = nf // fpt
    esplit = nsub // fgroups
    nshards = ncores * esplit
    eshard = e // nshards
    nch = eshard // chunk
    groups = chunk // lanes
    assert eshard % chunk == 0 and chunk % 128 == 0 and n % lanes == 0

    mesh = plsc.VectorSubcoreMesh(core_axis_name="c", subcore_axis_name="s")

    @functools.partial(
        pl.kernel,
        out_type=jax.ShapeDtypeStruct((nshards, nf, n), _f32),
        mesh=mesh,
        compiler_params=pltpu.CompilerParams(needs_layout_passes=False),
        scratch_types=[
            pltpu.VMEM((fpt, n), _f32),
            pltpu.VMEM((fpt, n), _f32),
            pltpu.VMEM((2, 2 * chunk), jnp.int32),
            pltpu.VMEM((2 * chunk,), _f32),
            pltpu.SemaphoreType.DMA,
            pltpu.SemaphoreType.DMA,
            pltpu.SemaphoreType.DMA,
        ],
    )
    def spmm(ht_hbm, ei_hbm, ew_hbm, out_hbm, src_v, acc_v, rc_b, w_b,
             sem0, sem1, sem_src):
        c = lax.axis_index("c")
        s = lax.axis_index("s")
        f0 = (s // esplit) * fpt
        shard = c * esplit + s % esplit
        ebase = shard * eshard

        src_cp = pltpu.async_copy(ht_hbm.at[pl.ds(f0, fpt)], src_v, sem_src)

        sems = (sem0, sem1)
        descs = {}

        def issue(ci, b):
            base = ebase + ci * chunk
            dst = pl.ds(b * chunk, chunk)
            descs[ci] = (
                pltpu.async_copy(ei_hbm.at[:, pl.ds(base, chunk)],
                                 rc_b.at[:, dst], sems[b]),
                pltpu.async_copy(ew_hbm.at[pl.ds(base, chunk)],
                                 w_b.at[dst], sems[b]),
            )

        issue(0, 0)
        if nch > 1:
            issue(1, 1)

        zeros16 = jnp.zeros((lanes,), _f32)

        @pl.loop(0, n // lanes, unroll=8)
        def _zero(i):
            for r in range(fpt):
                acc_v[r, pl.ds(i * lanes, lanes)] = zeros16

        src_cp.wait()

        ridx = [jnp.full((lanes,), r, jnp.int32) for r in range(fpt)]

        for ci in range(nch):
            b = ci % 2
            for d in descs.pop(ci):
                d.wait()

            @plsc.parallel_loop(0, groups, unroll=8)
            def _edges(k, b=b):
                sl = pl.ds(b * chunk + k * lanes, lanes)
                rv = rc_b[0, sl]
                cv = rc_b[1, sl]
                wv = w_b[sl]
                for r in range(fpt):
                    g = plsc.load_gather(src_v, [ridx[r], cv])
                    plsc.addupdate_scatter(acc_v, [ridx[r], rv], wv * g)

            if ci + 2 < nch:
                issue(ci + 2, b)

        pltpu.sync_copy(acc_v, out_hbm.at[shard, pl.ds(f0, fpt)])

    return spmm


def _dense1(x, w1, b1r):
    n, d = x.shape
    h1 = w1.shape[1]

    def body(x_ref, w_ref, b_ref, out_ref):
        acc = lax.dot_general(w_ref[...], x_ref[...],
                              (((0,), (1,)), ((), ())),
                              preferred_element_type=_f32)
        out_ref[...] = acc + b_ref[...].reshape(h1, 1)

    return pl.pallas_call(
        body,
        out_shape=jax.ShapeDtypeStruct((h1, n), _f32),
    )(x, w1, b1r)


def _dense2(p, wmu, bmur, wls, blsr):
    k, h1, n = p.shape
    h2 = wmu.shape[1]

    def body(p_ref, wmu_ref, bmu_ref, wls_ref, bls_ref, out_ref):
        h = jnp.maximum(sum(p_ref[i] for i in range(k)), 0.0)
        top = lax.dot_general(wmu_ref[...], h, (((0,), (0,)), ((), ())),
                              preferred_element_type=_f32)
        bot = lax.dot_general(wls_ref[...], h, (((0,), (0,)), ((), ())),
                              preferred_element_type=_f32)
        top = top + bmu_ref[...].reshape(h2, 1)
        bot = bot + bls_ref[...].reshape(h2, 1)
        out_ref[...] = jnp.concatenate([top, bot], axis=0)

    return pl.pallas_call(
        body,
        out_shape=jax.ShapeDtypeStruct((2 * h2, n), _f32),
    )(p, wmu, bmur, wls, blsr)


def _final(p, noise):
    k, nf, n = p.shape
    h2 = nf // 2

    def body(p_ref, noise_ref, z_ref, mu_ref, ls_ref):
        st = sum(p_ref[i] for i in range(k))
        eye = jax.lax.broadcasted_iota(jnp.int32, (nf, h2), 0)
        col = jax.lax.broadcasted_iota(jnp.int32, (nf, h2), 1)
        sel_mu = (eye == col).astype(_f32)
        sel_ls = (eye == col + h2).astype(_f32)
        mu = lax.dot_general(st, sel_mu, (((0,), (0,)), ((), ())),
                             preferred_element_type=_f32)
        ls = lax.dot_general(st, sel_ls, (((0,), (0,)), ((), ())),
                             preferred_element_type=_f32)
        mu_ref[...] = mu
        ls_ref[...] = ls
        z_ref[...] = mu + noise_ref[...] * jnp.exp(ls)

    out_sds = jax.ShapeDtypeStruct((n, h2), _f32)
    return pl.pallas_call(
        body,
        out_shape=[out_sds, out_sds, out_sds],
    )(p, noise)


def kernel(x, edge_index, edge_weight, noise, W1, b1, Wmu, bmu, Wls, bls):
    n, d = x.shape
    h1 = W1.shape[1]
    h2 = Wmu.shape[1]
    e = edge_weight.shape[0]

    spmm = _make_spmm(2 * h2, n, e)

    h0t = _dense1(x, W1, b1.reshape(1, h1))
    p1 = spmm(h0t, edge_index, edge_weight)
    pret = _dense2(p1, Wmu, bmu.reshape(1, h2), Wls, bls.reshape(1, h2))
    p2 = spmm(pret, edge_index, edge_weight)
    z, mu, logstd = _final(p2, noise)
    return (z, mu, logstd)


# speedup vs baseline: 1.2227x; 1.0344x over previous
import functools

import jax
import jax.numpy as jnp
from jax import lax
from jax.experimental import pallas as pl
from jax.experimental.pallas import tpu as pltpu
from jax.experimental.pallas import tpu_sc as plsc

_f32 = jnp.float32


def _make_spmm(nf, n, e, chunk=3200, fpt=4):
    ncores, nsub, lanes = 2, 16, 16
    fgroups = nf // fpt
    esplit = nsub // fgroups
    nshards = ncores * esplit
    eshard = e // nshards
    nch = eshard // chunk
    groups = chunk // lanes
    assert eshard % chunk == 0 and chunk % 128 == 0 and n % lanes == 0

    mesh = plsc.VectorSubcoreMesh(core_axis_name="c", subcore_axis_name="s")

    @functools.partial(
        pl.kernel,
        out_type=jax.ShapeDtypeStruct((nshards, nf, n), _f32),
        mesh=mesh,
        compiler_params=pltpu.CompilerParams(needs_layout_passes=False),
        scratch_types=[
            pltpu.VMEM((fpt, n), _f32),
            pltpu.VMEM((fpt, n), _f32),
            pltpu.VMEM((2, 2 * chunk), jnp.int32),
            pltpu.VMEM((2 * chunk,), _f32),
            pltpu.SemaphoreType.DMA,
            pltpu.SemaphoreType.DMA,
            pltpu.SemaphoreType.DMA,
        ],
    )
    def spmm(ht_hbm, ei_hbm, ew_hbm, out_hbm, src_v, acc_v, rc_b, w_b,
             sem0, sem1, sem_src):
        c = lax.axis_index("c")
        s = lax.axis_index("s")
        f0 = (s // esplit) * fpt
        shard = c * esplit + s % esplit
        ebase = shard * eshard

        src_cp = pltpu.async_copy(ht_hbm.at[pl.ds(f0, fpt)], src_v, sem_src)

        sems = (sem0, sem1)
        descs = {}

        def issue(ci, b):
            base = ebase + ci * chunk
            dst = pl.ds(b * chunk, chunk)
            descs[ci] = (
                pltpu.async_copy(ei_hbm.at[:, pl.ds(base, chunk)],
                                 rc_b.at[:, dst], sems[b]),
                pltpu.async_copy(ew_hbm.at[pl.ds(base, chunk)],
                                 w_b.at[dst], sems[b]),
            )

        issue(0, 0)
        if nch > 1:
            issue(1, 1)

        zeros16 = jnp.zeros((lanes,), _f32)

        @pl.loop(0, n // lanes, unroll=8)
        def _zero(i):
            for r in range(fpt):
                acc_v[r, pl.ds(i * lanes, lanes)] = zeros16

        src_cp.wait()

        ridx = [jnp.full((lanes,), r, jnp.int32) for r in range(fpt)]

        for ci in range(nch):
            b = ci % 2
            for d in descs.pop(ci):
                d.wait()

            @plsc.parallel_loop(0, groups, unroll=4)
            def _edges(k, b=b):
                sl = pl.ds(b * chunk + k * lanes, lanes)
                rv = rc_b[0, sl]
                cv = rc_b[1, sl]
                wv = w_b[sl]
                for r in range(fpt):
                    g = plsc.load_gather(src_v, [ridx[r], cv])
                    plsc.addupdate_scatter(acc_v, [ridx[r], rv], wv * g)

            if ci + 2 < nch:
                issue(ci + 2, b)

        pltpu.sync_copy(acc_v, out_hbm.at[shard, pl.ds(f0, fpt)])

    return spmm


def _dense1(x, w1, b1r):
    n, d = x.shape
    h1 = w1.shape[1]

    def body(x_ref, w_ref, b_ref, out_ref):
        acc = lax.dot_general(w_ref[...], x_ref[...],
                              (((0,), (1,)), ((), ())),
                              preferred_element_type=_f32)
        out_ref[...] = acc + b_ref[...].reshape(h1, 1)

    return pl.pallas_call(
        body,
        out_shape=jax.ShapeDtypeStruct((h1, n), _f32),
    )(x, w1, b1r)


def _dense2(p, wmu, bmur, wls, blsr):
    k, h1, n = p.shape
    h2 = wmu.shape[1]

    def body(p_ref, wmu_ref, bmu_ref, wls_ref, bls_ref, out_ref):
        h = jnp.maximum(sum(p_ref[i] for i in range(k)), 0.0)
        top = lax.dot_general(wmu_ref[...], h, (((0,), (0,)), ((), ())),
                              preferred_element_type=_f32)
        bot = lax.dot_general(wls_ref[...], h, (((0,), (0,)), ((), ())),
                              preferred_element_type=_f32)
        top = top + bmu_ref[...].reshape(h2, 1)
        bot = bot + bls_ref[...].reshape(h2, 1)
        out_ref[...] = jnp.concatenate([top, bot], axis=0)

    return pl.pallas_call(
        body,
        out_shape=jax.ShapeDtypeStruct((2 * h2, n), _f32),
    )(p, wmu, bmur, wls, blsr)


def _final(p, noise):
    k, nf, n = p.shape
    h2 = nf // 2

    def body(p_ref, noise_ref, z_ref, mu_ref, ls_ref):
        st = sum(p_ref[i] for i in range(k))
        eye = jax.lax.broadcasted_iota(jnp.int32, (nf, h2), 0)
        col = jax.lax.broadcasted_iota(jnp.int32, (nf, h2), 1)
        sel_mu = (eye == col).astype(_f32)
        sel_ls = (eye == col + h2).astype(_f32)
        mu = lax.dot_general(st, sel_mu, (((0,), (0,)), ((), ())),
                             preferred_element_type=_f32)
        ls = lax.dot_general(st, sel_ls, (((0,), (0,)), ((), ())),
                             preferred_element_type=_f32)
        mu_ref[...] = mu
        ls_ref[...] = ls
        z_ref[...] = mu + noise_ref[...] * jnp.exp(ls)

    out_sds = jax.ShapeDtypeStruct((n, h2), _f32)
    return pl.pallas_call(
        body,
        out_shape=[out_sds, out_sds, out_sds],
    )(p, noise)


def kernel(x, edge_index, edge_weight, noise, W1, b1, Wmu, bmu, Wls, bls):
    n, d = x.shape
    h1 = W1.shape[1]
    h2 = Wmu.shape[1]
    e = edge_weight.shape[0]

    spmm = _make_spmm(2 * h2, n, e)

    h0t = _dense1(x, W1, b1.reshape(1, h1))
    p1 = spmm(h0t, edge_index, edge_weight)
    pret = _dense2(p1, Wmu, bmu.reshape(1, h2), Wls, bls.reshape(1, h2))
    p2 = spmm(pret, edge_index, edge_weight)
    z, mu, logstd = _final(p2, noise)
    return (z, mu, logstd)


# speedup vs baseline: 1.2493x; 1.0217x over previous
import functools

import jax
import jax.numpy as jnp
from jax import lax
from jax.experimental import pallas as pl
from jax.experimental.pallas import tpu as pltpu
from jax.experimental.pallas import tpu_sc as plsc

_f32 = jnp.float32


def _make_spmm(nf, n, e, chunk=3200, fpt=4):
    ncores, nsub, lanes = 2, 16, 16
    fgroups = nf // fpt
    esplit = nsub // fgroups
    nshards = ncores * esplit
    eshard = e // nshards
    nch = eshard // chunk
    groups = chunk // lanes
    assert eshard % chunk == 0 and chunk % 128 == 0 and n % lanes == 0

    mesh = plsc.VectorSubcoreMesh(core_axis_name="c", subcore_axis_name="s")

    @functools.partial(
        pl.kernel,
        out_type=jax.ShapeDtypeStruct((nshards, nf, n), _f32),
        mesh=mesh,
        compiler_params=pltpu.CompilerParams(needs_layout_passes=False),
        scratch_types=[
            pltpu.VMEM((fpt, n), _f32),
            pltpu.VMEM((fpt, n), _f32),
            pltpu.VMEM((2, 2 * chunk), jnp.int32),
            pltpu.VMEM((2 * chunk,), _f32),
            pltpu.SemaphoreType.DMA,
            pltpu.SemaphoreType.DMA,
            pltpu.SemaphoreType.DMA,
        ],
    )
    def spmm(ht_hbm, ei_hbm, ew_hbm, out_hbm, src_v, acc_v, rc_b, w_b,
             sem0, sem1, sem_src):
        c = lax.axis_index("c")
        s = lax.axis_index("s")
        f0 = (s // esplit) * fpt
        shard = c * esplit + s % esplit
        ebase = shard * eshard

        src_cp = pltpu.async_copy(ht_hbm.at[pl.ds(f0, fpt)], src_v, sem_src)

        sems = (sem0, sem1)
        descs = {}

        def issue(ci, b):
            base = ebase + ci * chunk
            dst = pl.ds(b * chunk, chunk)
            descs[ci] = (
                pltpu.async_copy(ei_hbm.at[:, pl.ds(base, chunk)],
                                 rc_b.at[:, dst], sems[b]),
                pltpu.async_copy(ew_hbm.at[pl.ds(base, chunk)],
                                 w_b.at[dst], sems[b]),
            )

        issue(0, 0)
        if nch > 1:
            issue(1, 1)

        zeros16 = jnp.zeros((lanes,), _f32)

        @pl.loop(0, n // lanes, unroll=8)
        def _zero(i):
            for r in range(fpt):
                acc_v[r, pl.ds(i * lanes, lanes)] = zeros16

        src_cp.wait()

        ridx = [jnp.full((lanes,), r, jnp.int32) for r in range(fpt)]

        for ci in range(nch):
            b = ci % 2
            for d in descs.pop(ci):
                d.wait()

            @plsc.parallel_loop(0, groups, unroll=2)
            def _edges(k, b=b):
                sl = pl.ds(b * chunk + k * lanes, lanes)
                rv = rc_b[0, sl]
                cv = rc_b[1, sl]
                wv = w_b[sl]
                for r in range(fpt):
                    g = plsc.load_gather(src_v, [ridx[r], cv])
                    plsc.addupdate_scatter(acc_v, [ridx[r], rv], wv * g)

            if ci + 2 < nch:
                issue(ci + 2, b)

        pltpu.sync_copy(acc_v, out_hbm.at[shard, pl.ds(f0, fpt)])

    return spmm


def _dense1(x, w1, b1r):
    n, d = x.shape
    h1 = w1.shape[1]

    def body(x_ref, w_ref, b_ref, out_ref):
        acc = lax.dot_general(w_ref[...], x_ref[...],
                              (((0,), (1,)), ((), ())),
                              preferred_element_type=_f32)
        out_ref[...] = acc + b_ref[...].reshape(h1, 1)

    return pl.pallas_call(
        body,
        out_shape=jax.ShapeDtypeStruct((h1, n), _f32),
    )(x, w1, b1r)


def _dense2(p, wmu, bmur, wls, blsr):
    k, h1, n = p.shape
    h2 = wmu.shape[1]

    def body(p_ref, wmu_ref, bmu_ref, wls_ref, bls_ref, out_ref):
        h = jnp.maximum(sum(p_ref[i] for i in range(k)), 0.0)
        top = lax.dot_general(wmu_ref[...], h, (((0,), (0,)), ((), ())),
                              preferred_element_type=_f32)
        bot = lax.dot_general(wls_ref[...], h, (((0,), (0,)), ((), ())),
                              preferred_element_type=_f32)
        top = top + bmu_ref[...].reshape(h2, 1)
        bot = bot + bls_ref[...].reshape(h2, 1)
        out_ref[...] = jnp.concatenate([top, bot], axis=0)

    return pl.pallas_call(
        body,
        out_shape=jax.ShapeDtypeStruct((2 * h2, n), _f32),
    )(p, wmu, bmur, wls, blsr)


def _final(p, noise):
    k, nf, n = p.shape
    h2 = nf // 2

    def body(p_ref, noise_ref, z_ref, mu_ref, ls_ref):
        st = sum(p_ref[i] for i in range(k))
        eye = jax.lax.broadcasted_iota(jnp.int32, (nf, h2), 0)
        col = jax.lax.broadcasted_iota(jnp.int32, (nf, h2), 1)
        sel_mu = (eye == col).astype(_f32)
        sel_ls = (eye == col + h2).astype(_f32)
        mu = lax.dot_general(st, sel_mu, (((0,), (0,)), ((), ())),
                             preferred_element_type=_f32)
        ls = lax.dot_general(st, sel_ls, (((0,), (0,)), ((), ())),
                             preferred_element_type=_f32)
        mu_ref[...] = mu
        ls_ref[...] = ls
        z_ref[...] = mu + noise_ref[...] * jnp.exp(ls)

    out_sds = jax.ShapeDtypeStruct((n, h2), _f32)
    return pl.pallas_call(
        body,
        out_shape=[out_sds, out_sds, out_sds],
    )(p, noise)


def kernel(x, edge_index, edge_weight, noise, W1, b1, Wmu, bmu, Wls, bls):
    n, d = x.shape
    h1 = W1.shape[1]
    h2 = Wmu.shape[1]
    e = edge_weight.shape[0]

    spmm = _make_spmm(2 * h2, n, e)

    h0t = _dense1(x, W1, b1.reshape(1, h1))
    p1 = spmm(h0t, edge_index, edge_weight)
    pret = _dense2(p1, Wmu, bmu.reshape(1, h2), Wls, bls.reshape(1, h2))
    p2 = spmm(pret, edge_index, edge_weight)
    z, mu, logstd = _final(p2, noise)
    return (z, mu, logstd)


# speedup vs baseline: 1.2550x; 1.0046x over previous
import functools

import jax
import jax.numpy as jnp
from jax import lax
from jax.experimental import pallas as pl
from jax.experimental.pallas import tpu as pltpu
from jax.experimental.pallas import tpu_sc as plsc

_f32 = jnp.float32


def _make_spmm(nf, n, e, chunk=3200, fpt=4):
    ncores, nsub, lanes = 2, 16, 16
    fgroups = nf // fpt
    esplit = nsub // fgroups
    nshards = ncores * esplit
    eshard = e // nshards
    nch = eshard // chunk
    groups = chunk // lanes
    assert eshard % chunk == 0 and chunk % 128 == 0 and n % lanes == 0

    mesh = plsc.VectorSubcoreMesh(core_axis_name="c", subcore_axis_name="s")

    @functools.partial(
        pl.kernel,
        out_type=jax.ShapeDtypeStruct((nshards, nf, n), _f32),
        mesh=mesh,
        compiler_params=pltpu.CompilerParams(needs_layout_passes=False),
        scratch_types=[
            pltpu.VMEM((fpt, n), _f32),
            pltpu.VMEM((fpt, n), _f32),
            pltpu.VMEM((2, 2 * chunk), jnp.int32),
            pltpu.VMEM((2 * chunk,), _f32),
            pltpu.SemaphoreType.DMA,
            pltpu.SemaphoreType.DMA,
            pltpu.SemaphoreType.DMA,
        ],
    )
    def spmm(ht_hbm, ei_hbm, ew_hbm, out_hbm, src_v, acc_v, rc_b, w_b,
             sem0, sem1, sem_src):
        c = lax.axis_index("c")
        s = lax.axis_index("s")
        f0 = (s // esplit) * fpt
        shard = c * esplit + s % esplit
        ebase = shard * eshard

        src_cp = pltpu.async_copy(ht_hbm.at[pl.ds(f0, fpt)], src_v, sem_src)

        sems = (sem0, sem1)
        descs = {}

        def issue(ci, b):
            base = ebase + ci * chunk
            dst = pl.ds(b * chunk, chunk)
            descs[ci] = (
                pltpu.async_copy(ei_hbm.at[:, pl.ds(base, chunk)],
                                 rc_b.at[:, dst], sems[b]),
                pltpu.async_copy(ew_hbm.at[pl.ds(base, chunk)],
                                 w_b.at[dst], sems[b]),
            )

        issue(0, 0)
        if nch > 1:
            issue(1, 1)

        zeros16 = jnp.zeros((lanes,), _f32)

        @pl.loop(0, n // lanes, unroll=8)
        def _zero(i):
            for r in range(fpt):
                acc_v[r, pl.ds(i * lanes, lanes)] = zeros16

        src_cp.wait()

        ridx = [jnp.full((lanes,), r, jnp.int32) for r in range(fpt)]

        for ci in range(nch):
            b = ci % 2
            for d in descs.pop(ci):
                d.wait()

            @plsc.parallel_loop(0, groups, unroll=1)
            def _edges(k, b=b):
                sl = pl.ds(b * chunk + k * lanes, lanes)
                rv = rc_b[0, sl]
                cv = rc_b[1, sl]
                wv = w_b[sl]
                for r in range(fpt):
                    g = plsc.load_gather(src_v, [ridx[r], cv])
                    plsc.addupdate_scatter(acc_v, [ridx[r], rv], wv * g)

            if ci + 2 < nch:
                issue(ci + 2, b)

        pltpu.sync_copy(acc_v, out_hbm.at[shard, pl.ds(f0, fpt)])

    return spmm


def _dense1(x, w1, b1r):
    n, d = x.shape
    h1 = w1.shape[1]

    def body(x_ref, w_ref, b_ref, out_ref):
        acc = lax.dot_general(w_ref[...], x_ref[...],
                              (((0,), (1,)), ((), ())),
                              preferred_element_type=_f32)
        out_ref[...] = acc + b_ref[...].reshape(h1, 1)

    return pl.pallas_call(
        body,
        out_shape=jax.ShapeDtypeStruct((h1, n), _f32),
    )(x, w1, b1r)


def _dense2(p, wmu, bmur, wls, blsr):
    k, h1, n = p.shape
    h2 = wmu.shape[1]

    def body(p_ref, wmu_ref, bmu_ref, wls_ref, bls_ref, out_ref):
        h = jnp.maximum(sum(p_ref[i] for i in range(k)), 0.0)
        top = lax.dot_general(wmu_ref[...], h, (((0,), (0,)), ((), ())),
                              preferred_element_type=_f32)
        bot = lax.dot_general(wls_ref[...], h, (((0,), (0,)), ((), ())),
                              preferred_element_type=_f32)
        top = top + bmu_ref[...].reshape(h2, 1)
        bot = bot + bls_ref[...].reshape(h2, 1)
        out_ref[...] = jnp.concatenate([top, bot], axis=0)

    return pl.pallas_call(
        body,
        out_shape=jax.ShapeDtypeStruct((2 * h2, n), _f32),
    )(p, wmu, bmur, wls, blsr)


def _final(p, noise):
    k, nf, n = p.shape
    h2 = nf // 2

    def body(p_ref, noise_ref, z_ref, mu_ref, ls_ref):
        st = sum(p_ref[i] for i in range(k))
        eye = jax.lax.broadcasted_iota(jnp.int32, (nf, h2), 0)
        col = jax.lax.broadcasted_iota(jnp.int32, (nf, h2), 1)
        sel_mu = (eye == col).astype(_f32)
        sel_ls = (eye == col + h2).astype(_f32)
        mu = lax.dot_general(st, sel_mu, (((0,), (0,)), ((), ())),
                             preferred_element_type=_f32)
        ls = lax.dot_general(st, sel_ls, (((0,), (0,)), ((), ())),
                             preferred_element_type=_f32)
        mu_ref[...] = mu
        ls_ref[...] = ls
        z_ref[...] = mu + noise_ref[...] * jnp.exp(ls)

    out_sds = jax.ShapeDtypeStruct((n, h2), _f32)
    return pl.pallas_call(
        body,
        out_shape=[out_sds, out_sds, out_sds],
    )(p, noise)


def kernel(x, edge_index, edge_weight, noise, W1, b1, Wmu, bmu, Wls, bls):
    n, d = x.shape
    h1 = W1.shape[1]
    h2 = Wmu.shape[1]
    e = edge_weight.shape[0]

    spmm = _make_spmm(2 * h2, n, e)

    h0t = _dense1(x, W1, b1.reshape(1, h1))
    p1 = spmm(h0t, edge_index, edge_weight)
    pret = _dense2(p1, Wmu, bmu.reshape(1, h2), Wls, bls.reshape(1, h2))
    p2 = spmm(pret, edge_index, edge_weight)
    z, mu, logstd = _final(p2, noise)
    return (z, mu, logstd)
